# Initial kernel scaffold; baseline (speedup 1.0000x reference)
import jax
import jax.numpy as jnp
from jax.experimental import pallas as pl


def kernel(x, edge_index, W1, b1, Wmu, bmu, Wlv, blv):
    raise NotImplementedError("write your pallas kernel here")


# speedup vs baseline: 20.1656x; 20.1656x over previous
import functools
import math

import jax
import jax.numpy as jnp
from jax import lax
from jax.experimental import pallas as pl
from jax.experimental.pallas import tpu as pltpu
from jax.experimental.pallas import tpu_sc as plsc

_NC = 2
_NS = 16
_K = 128
_MBLK = 1024


def _deg_kernel(NP, nb):
    NW = _NC * _NS
    SN = NP // _NS
    mesh = plsc.VectorSubcoreMesh(core_axis_name="c", subcore_axis_name="s")

    @functools.partial(
        pl.kernel,
        out_type=jax.ShapeDtypeStruct((_NC, NP), jnp.float32),
        mesh=mesh,
        scratch_types=[
            pltpu.VMEM((nb, _K), jnp.int32),
            pltpu.VMEM((_K,), jnp.float32),
            pltpu.VMEM((SN,), jnp.float32),
            pltpu.VMEM_SHARED((NP,), jnp.float32),
        ],
    )
    def k(dst_hbm, out_hbm, didx, ones, zbuf, acc):
        c = lax.axis_index("c")
        s = lax.axis_index("s")
        wid = c * _NS + s
        pltpu.sync_copy(dst_hbm.at[wid], didx)
        for j in range(_K // 16):
            ones[pl.ds(j * 16, 16)] = jnp.ones((16,), jnp.float32)

        def zb(i, t):
            zbuf[pl.ds(i * 16, 16)] = jnp.zeros((16,), jnp.float32)
            return t

        lax.fori_loop(0, SN // 16, zb, 0)
        pltpu.sync_copy(zbuf, acc.at[pl.ds(s * SN, SN)])
        plsc.subcore_barrier()

        def body(b, t):
            pltpu.sync_copy(ones, acc.at[didx.at[b]], add=True)
            return t

        lax.fori_loop(0, nb, body, 0)
        plsc.subcore_barrier()
        pltpu.sync_copy(acc.at[pl.ds(s * SN, SN)], out_hbm.at[c, pl.ds(s * SN, SN)])

    return k


def _agg_kernel(NP, D, nb):
    NW = _NC * _NS
    SN = NP // _NS
    mesh = plsc.VectorSubcoreMesh(core_axis_name="c", subcore_axis_name="s")

    @functools.partial(
        pl.kernel,
        out_type=jax.ShapeDtypeStruct((_NC, NP, D), jnp.float32),
        mesh=mesh,
        scratch_types=[
            pltpu.VMEM((nb, _K), jnp.int32),
            pltpu.VMEM((nb, _K), jnp.int32),
            pltpu.VMEM((_K, D), jnp.float32),
            pltpu.VMEM_SHARED((NP, D), jnp.float32),
            pltpu.SemaphoreType.DMA,
        ],
    )
    def k(u_hbm, src_hbm, dst_hbm, out_hbm, sidx, didx, rows, acc, sem):
        c = lax.axis_index("c")
        s = lax.axis_index("s")
        wid = c * _NS + s
        pltpu.sync_copy(src_hbm.at[wid], sidx)
        pltpu.sync_copy(dst_hbm.at[wid], didx)

        def zr(i, t):
            for j in range(D // 16):
                rows[i, pl.ds(j * 16, 16)] = jnp.zeros((16,), jnp.float32)
            return t

        lax.fori_loop(0, _K, zr, 0)
        for t in range(SN // _K):
            pltpu.sync_copy(rows, acc.at[pl.ds(s * SN + t * _K, _K)])
        plsc.subcore_barrier()

        def body(b, t):
            pltpu.async_copy(u_hbm.at[sidx.at[b]], rows, sem).wait()
            pltpu.sync_copy(rows, acc.at[didx.at[b]], add=True)
            return t

        lax.fori_loop(0, nb, body, 0)
        plsc.subcore_barrier()
        for t in range(SN // _K):
            pltpu.sync_copy(
                acc.at[pl.ds(s * SN + t * _K, _K)],
                out_hbm.at[c, pl.ds(s * SN + t * _K, _K)],
            )

    return k


def _dinv(degp_ref):
    deg = degp_ref[0, :] + degp_ref[1, :]
    return jnp.where(deg > 0, lax.rsqrt(deg), 0.0)


def _mm1_body(degp_ref, x_ref, w_ref, u_ref):
    dinv = _dinv(degp_ref)
    xw = jnp.dot(x_ref[...], w_ref[...], preferred_element_type=jnp.float32,
                 precision=lax.Precision.HIGHEST)
    u_ref[...] = xw * dinv[:, None]


def _mid_body(degp_ref, p_ref, b_ref, w_ref, u_ref):
    dinv = _dinv(degp_ref)
    ssum = p_ref[0] + p_ref[1]
    h = jnp.maximum(ssum * dinv[:, None] + b_ref[...], 0.0)
    hw = jnp.dot(h, w_ref[...], preferred_element_type=jnp.float32,
                 precision=lax.Precision.HIGHEST)
    u_ref[...] = hw * dinv[:, None]


def _fin_body(degp_ref, q_ref, b_ref, o_ref):
    dinv = _dinv(degp_ref)
    qsum = q_ref[0] + q_ref[1]
    o_ref[...] = qsum * dinv[:, None] + b_ref[...]


def _mm1(NP, DIN, DH):
    return pl.pallas_call(
        _mm1_body,
        grid=(NP // _MBLK,),
        in_specs=[
            pl.BlockSpec((_NC, _MBLK), lambda i: (0, i)),
            pl.BlockSpec((_MBLK, DIN), lambda i: (i, 0)),
            pl.BlockSpec((DIN, DH), lambda i: (0, 0)),
        ],
        out_specs=pl.BlockSpec((_MBLK, DH), lambda i: (i, 0)),
        out_shape=jax.ShapeDtypeStruct((NP, DH), jnp.float32),
    )


def _mid(NP, DH, D2):
    return pl.pallas_call(
        _mid_body,
        grid=(NP // _MBLK,),
        in_specs=[
            pl.BlockSpec((_NC, _MBLK), lambda i: (0, i)),
            pl.BlockSpec((_NC, _MBLK, DH), lambda i: (0, i, 0)),
            pl.BlockSpec((1, DH), lambda i: (0, 0)),
            pl.BlockSpec((DH, D2), lambda i: (0, 0)),
        ],
        out_specs=pl.BlockSpec((_MBLK, D2), lambda i: (i, 0)),
        out_shape=jax.ShapeDtypeStruct((NP, D2), jnp.float32),
    )


def _fin(NP, D2):
    return pl.pallas_call(
        _fin_body,
        grid=(NP // _MBLK,),
        in_specs=[
            pl.BlockSpec((_NC, _MBLK), lambda i: (0, i)),
            pl.BlockSpec((_NC, _MBLK, D2), lambda i: (0, i, 0)),
            pl.BlockSpec((1, D2), lambda i: (0, 0)),
        ],
        out_specs=pl.BlockSpec((_MBLK, D2), lambda i: (i, 0)),
        out_shape=jax.ShapeDtypeStruct((NP, D2), jnp.float32),
    )


def _impl(x, edge_index, W1, b1, Wmu, bmu, Wlv, blv):
    N, DIN = x.shape
    DH = W1.shape[1]
    DOUT = Wmu.shape[1]
    D2 = 2 * DOUT
    E = edge_index.shape[1]
    NW = _NC * _NS

    NP = math.ceil((N + 1) / (_NS * _K)) * (_NS * _K)
    NP = math.ceil(NP / _MBLK) * _MBLK
    Etot = E + N
    nb = math.ceil(Etot / (NW * _K))
    EP = NW * nb * _K

    sl = jnp.arange(N, dtype=jnp.int32)
    pad = jnp.full((EP - Etot,), N, dtype=jnp.int32)
    src2 = jnp.concatenate([edge_index[0], sl, pad]).reshape(NW, nb, _K)
    dst2 = jnp.concatenate([edge_index[1], sl, pad]).reshape(NW, nb, _K)
    xp = jnp.zeros((NP, DIN), jnp.float32).at[:N].set(x)

    degp = _deg_kernel(NP, nb)(dst2)
    u1 = _mm1(NP, DIN, DH)(degp, xp, W1)
    P = _agg_kernel(NP, DH, nb)(u1, src2, dst2)
    Wcat = jnp.concatenate([Wmu, Wlv], axis=1)
    bcat = jnp.concatenate([bmu, blv]).reshape(1, D2)
    u2 = _mid(NP, DH, D2)(degp, P, b1.reshape(1, DH), Wcat)
    Q = _agg_kernel(NP, D2, nb)(u2, src2, dst2)
    out = _fin(NP, D2)(degp, Q, bcat)
    return out[:N, :DOUT], out[:N, DOUT:]


_jimpl = jax.jit(_impl)


def kernel(x, edge_index, W1, b1, Wmu, bmu, Wlv, blv):
    return _jimpl(x, edge_index, W1, b1, Wmu, bmu, Wlv, blv)


# speedup vs baseline: 27.4421x; 1.3608x over previous
import functools
import math

import jax
import jax.numpy as jnp
from jax import lax
from jax.experimental import pallas as pl
from jax.experimental.pallas import tpu as pltpu
from jax.experimental.pallas import tpu_sc as plsc

_NC = 2
_NS = 16
_K = 64
_MBLK = 1024


def _deg_kernel(NP, nb):
    NW = _NC * _NS
    SN = NP // _NS
    mesh = plsc.VectorSubcoreMesh(core_axis_name="c", subcore_axis_name="s")

    @functools.partial(
        pl.kernel,
        out_type=jax.ShapeDtypeStruct((_NC, NP), jnp.float32),
        mesh=mesh,
        scratch_types=[
            pltpu.VMEM((nb, _K), jnp.int32),
            pltpu.VMEM((_K,), jnp.float32),
            pltpu.VMEM((SN,), jnp.float32),
            pltpu.VMEM_SHARED((NP,), jnp.float32),
        ],
    )
    def k(dst_hbm, out_hbm, didx, ones, zbuf, acc):
        c = lax.axis_index("c")
        s = lax.axis_index("s")
        wid = c * _NS + s
        pltpu.sync_copy(dst_hbm.at[wid], didx)
        for j in range(_K // 16):
            ones[pl.ds(j * 16, 16)] = jnp.ones((16,), jnp.float32)

        def zb(i, t):
            zbuf[pl.ds(i * 16, 16)] = jnp.zeros((16,), jnp.float32)
            return t

        lax.fori_loop(0, SN // 16, zb, 0)
        pltpu.sync_copy(zbuf, acc.at[pl.ds(s * SN, SN)])
        plsc.subcore_barrier()

        def body(b, t):
            pltpu.sync_copy(ones, acc.at[didx.at[b]], add=True)
            return t

        lax.fori_loop(0, nb, body, 0)
        plsc.subcore_barrier()
        pltpu.sync_copy(acc.at[pl.ds(s * SN, SN)], out_hbm.at[c, pl.ds(s * SN, SN)])

    return k


def _agg_kernel(NP, D, noct):
    NW = _NC * _NS
    SN = NP // _NS
    mesh = plsc.VectorSubcoreMesh(core_axis_name="c", subcore_axis_name="s")

    @functools.partial(
        pl.kernel,
        out_type=jax.ShapeDtypeStruct((_NC, NP, D), jnp.float32),
        mesh=mesh,
        scratch_types=[
            pltpu.VMEM((8, _K), jnp.int32),
            pltpu.VMEM((8, _K), jnp.int32),
            pltpu.VMEM((_K,), jnp.int32),
            pltpu.VMEM((2, _K, D), jnp.float32),
            pltpu.VMEM_SHARED((NP, D), jnp.float32),
            pltpu.SemaphoreType.DMA,
            pltpu.SemaphoreType.DMA,
            pltpu.SemaphoreType.DMA,
        ],
    )
    def k(u_hbm, src_hbm, dst_hbm, out_hbm, sidx8, didx8, dtail, rows, acc,
          sg0, sg1, ss):
        c = lax.axis_index("c")
        s = lax.axis_index("s")
        wid = c * _NS + s

        def zr(i, t):
            for j in range(D // 16):
                rows[0, i, pl.ds(j * 16, 16)] = jnp.zeros((16,), jnp.float32)
            return t

        lax.fori_loop(0, _K, zr, 0)
        for t in range(SN // _K):
            pltpu.sync_copy(rows.at[0], acc.at[pl.ds(s * SN + t * _K, _K)])
        plsc.subcore_barrier()

        gsem = (sg0, sg1)

        def iload(o):
            pltpu.sync_copy(src_hbm.at[wid, pl.ds(o * 8, 8)], sidx8)
            pltpu.sync_copy(dst_hbm.at[wid, pl.ds(o * 8, 8)], didx8)

        def g_start(q, j):
            pltpu.async_copy(u_hbm.at[sidx8.at[q]], rows.at[j], gsem[j])

        def g_wait(j):
            pltpu.make_async_copy(u_hbm.at[pl.ds(0, _K)], rows.at[j],
                                  gsem[j]).wait()

        def s_wait():
            pltpu.make_async_copy(u_hbm.at[pl.ds(0, _K)], rows.at[1], ss).wait()

        iload(0)
        g_start(0, 0)

        def body(o, t):
            for q in range(8):
                j = q % 2
                if q + 1 < 8:
                    if q == 0:
                        @pl.when(o > 0)
                        def _():
                            s_wait()
                    g_start(q + 1, 1 - j)
                g_wait(j)
                if q < 7:
                    pltpu.sync_copy(rows.at[j], acc.at[didx8.at[q]], add=True)
                else:
                    for v in range(_K // 16):
                        dtail[pl.ds(v * 16, 16)] = didx8[q, pl.ds(v * 16, 16)]
                    pltpu.async_copy(rows.at[j], acc.at[dtail], ss, add=True)
            o1 = jnp.minimum(o + 1, noct - 1)
            iload(o1)
            g_start(0, 0)
            return t

        lax.fori_loop(0, noct, body, 0)
        s_wait()
        g_wait(0)
        plsc.subcore_barrier()
        for t in range(SN // _K):
            pltpu.sync_copy(
                acc.at[pl.ds(s * SN + t * _K, _K)],
                out_hbm.at[c, pl.ds(s * SN + t * _K, _K)],
            )

    return k


def _dinv(degp_ref):
    deg = degp_ref[0, :] + degp_ref[1, :]
    return jnp.where(deg > 0, lax.rsqrt(deg), 0.0)


def _mm1_body(degp_ref, x_ref, w_ref, u_ref):
    dinv = _dinv(degp_ref)
    xw = jnp.dot(x_ref[...], w_ref[...], preferred_element_type=jnp.float32,
                 precision=lax.Precision.HIGHEST)
    u_ref[...] = xw * dinv[:, None]


def _mid_body(degp_ref, p_ref, b_ref, w_ref, u_ref):
    dinv = _dinv(degp_ref)
    ssum = p_ref[0] + p_ref[1]
    h = jnp.maximum(ssum * dinv[:, None] + b_ref[...], 0.0)
    hw = jnp.dot(h, w_ref[...], preferred_element_type=jnp.float32,
                 precision=lax.Precision.HIGHEST)
    u_ref[...] = hw * dinv[:, None]


def _fin_body(degp_ref, q_ref, b_ref, o_ref):
    dinv = _dinv(degp_ref)
    qsum = q_ref[0] + q_ref[1]
    o_ref[...] = qsum * dinv[:, None] + b_ref[...]


def _mm1(NP, DIN, DH):
    return pl.pallas_call(
        _mm1_body,
        grid=(NP // _MBLK,),
        in_specs=[
            pl.BlockSpec((_NC, _MBLK), lambda i: (0, i)),
            pl.BlockSpec((_MBLK, DIN), lambda i: (i, 0)),
            pl.BlockSpec((DIN, DH), lambda i: (0, 0)),
        ],
        out_specs=pl.BlockSpec((_MBLK, DH), lambda i: (i, 0)),
        out_shape=jax.ShapeDtypeStruct((NP, DH), jnp.float32),
    )


def _mid(NP, DH, D2):
    return pl.pallas_call(
        _mid_body,
        grid=(NP // _MBLK,),
        in_specs=[
            pl.BlockSpec((_NC, _MBLK), lambda i: (0, i)),
            pl.BlockSpec((_NC, _MBLK, DH), lambda i: (0, i, 0)),
            pl.BlockSpec((1, DH), lambda i: (0, 0)),
            pl.BlockSpec((DH, D2), lambda i: (0, 0)),
        ],
        out_specs=pl.BlockSpec((_MBLK, D2), lambda i: (i, 0)),
        out_shape=jax.ShapeDtypeStruct((NP, D2), jnp.float32),
    )


def _fin(NP, D2):
    return pl.pallas_call(
        _fin_body,
        grid=(NP // _MBLK,),
        in_specs=[
            pl.BlockSpec((_NC, _MBLK), lambda i: (0, i)),
            pl.BlockSpec((_NC, _MBLK, D2), lambda i: (0, i, 0)),
            pl.BlockSpec((1, D2), lambda i: (0, 0)),
        ],
        out_specs=pl.BlockSpec((_MBLK, D2), lambda i: (i, 0)),
        out_shape=jax.ShapeDtypeStruct((NP, D2), jnp.float32),
    )


def _impl(x, edge_index, W1, b1, Wmu, bmu, Wlv, blv):
    N, DIN = x.shape
    DH = W1.shape[1]
    DOUT = Wmu.shape[1]
    D2 = 2 * DOUT
    E = edge_index.shape[1]
    NW = _NC * _NS

    NP = math.ceil((N + 1) / (_NS * _K)) * (_NS * _K)
    NP = math.ceil(NP / _MBLK) * _MBLK
    Etot = E + N
    C = math.ceil(Etot / NW)
    nb8 = math.ceil(C / _K)
    nb8 = math.ceil(nb8 / 8) * 8
    SLAB = nb8 * _K
    noct = nb8 // 8

    def dump(n, off):
        return (N + (off + jnp.arange(n, dtype=jnp.int32)) % (NP - N)).astype(
            jnp.int32)

    def slabify(flat):
        a = jnp.concatenate([flat, dump(NW * C - Etot, 0)]).reshape(NW, C)
        b = dump(NW * (SLAB - C), 7).reshape(NW, SLAB - C)
        return jnp.concatenate([a, b], axis=1).reshape(NW, nb8, _K)

    sl = jnp.arange(N, dtype=jnp.int32)
    src2 = slabify(jnp.concatenate([edge_index[0], sl]))
    dst2 = slabify(jnp.concatenate([edge_index[1], sl]))
    xp = jnp.zeros((NP, DIN), jnp.float32).at[:N].set(x)

    degp = _deg_kernel(NP, nb8)(dst2)
    u1 = _mm1(NP, DIN, DH)(degp, xp, W1)
    P = _agg_kernel(NP, DH, noct)(u1, src2, dst2)
    Wcat = jnp.concatenate([Wmu, Wlv], axis=1)
    bcat = jnp.concatenate([bmu, blv]).reshape(1, D2)
    u2 = _mid(NP, DH, D2)(degp, P, b1.reshape(1, DH), Wcat)
    Q = _agg_kernel(NP, D2, noct)(u2, src2, dst2)
    out = _fin(NP, D2)(degp, Q, bcat)
    return out[:N, :DOUT], out[:N, DOUT:]


_jimpl = jax.jit(_impl)


def kernel(x, edge_index, W1, b1, Wmu, bmu, Wlv, blv):
    return _jimpl(x, edge_index, W1, b1, Wmu, bmu, Wlv, blv)


# speedup vs baseline: 31.0409x; 1.1311x over previous
import functools
import math

import jax
import jax.numpy as jnp
from jax import lax
from jax.experimental import pallas as pl
from jax.experimental.pallas import tpu as pltpu
from jax.experimental.pallas import tpu_sc as plsc

_NC = 2
_NS = 16
_K = 128
_MBLK = 1024


def _deg_kernel(NP, nb):
    NW = _NC * _NS
    SN = NP // _NS
    mesh = plsc.VectorSubcoreMesh(core_axis_name="c", subcore_axis_name="s")

    @functools.partial(
        pl.kernel,
        out_type=jax.ShapeDtypeStruct((_NC, NP), jnp.float32),
        mesh=mesh,
        scratch_types=[
            pltpu.VMEM((nb, _K), jnp.int32),
            pltpu.VMEM((_K,), jnp.float32),
            pltpu.VMEM((SN,), jnp.float32),
            pltpu.VMEM_SHARED((NP,), jnp.float32),
        ],
    )
    def k(dst_hbm, out_hbm, didx, ones, zbuf, acc):
        c = lax.axis_index("c")
        s = lax.axis_index("s")
        wid = c * _NS + s
        pltpu.sync_copy(dst_hbm.at[wid], didx)
        for j in range(_K // 16):
            ones[pl.ds(j * 16, 16)] = jnp.ones((16,), jnp.float32)

        def zb(i, t):
            zbuf[pl.ds(i * 16, 16)] = jnp.zeros((16,), jnp.float32)
            return t

        lax.fori_loop(0, SN // 16, zb, 0)
        pltpu.sync_copy(zbuf, acc.at[pl.ds(s * SN, SN)])
        plsc.subcore_barrier()

        def body(b, t):
            pltpu.sync_copy(ones, acc.at[didx.at[b]], add=True)
            return t

        lax.fori_loop(0, nb, body, 0)
        plsc.subcore_barrier()
        pltpu.sync_copy(acc.at[pl.ds(s * SN, SN)], out_hbm.at[c, pl.ds(s * SN, SN)])

    return k


def _agg_kernel(NP, D, noct):
    NW = _NC * _NS
    SN = NP // _NS
    mesh = plsc.VectorSubcoreMesh(core_axis_name="c", subcore_axis_name="s")

    @functools.partial(
        pl.kernel,
        out_type=jax.ShapeDtypeStruct((_NC, NP, D), jnp.float32),
        mesh=mesh,
        scratch_types=[
            pltpu.VMEM((8, _K), jnp.int32),
            pltpu.VMEM((8, _K), jnp.int32),
            pltpu.VMEM((_K,), jnp.int32),
            pltpu.VMEM((2, _K, D), jnp.float32),
            pltpu.VMEM_SHARED((NP, D), jnp.float32),
            pltpu.SemaphoreType.DMA,
            pltpu.SemaphoreType.DMA,
            pltpu.SemaphoreType.DMA,
        ],
    )
    def k(u_hbm, src_hbm, dst_hbm, out_hbm, sidx8, didx8, dtail, rows, acc,
          sg0, sg1, ss):
        c = lax.axis_index("c")
        s = lax.axis_index("s")
        wid = c * _NS + s

        def zr(i, t):
            for j in range(D // 16):
                rows[0, i, pl.ds(j * 16, 16)] = jnp.zeros((16,), jnp.float32)
            return t

        lax.fori_loop(0, _K, zr, 0)
        for t in range(SN // _K):
            pltpu.sync_copy(rows.at[0], acc.at[pl.ds(s * SN + t * _K, _K)])
        plsc.subcore_barrier()

        gsem = (sg0, sg1)

        def iload(o):
            pltpu.sync_copy(src_hbm.at[wid, pl.ds(o * 8, 8)], sidx8)
            pltpu.sync_copy(dst_hbm.at[wid, pl.ds(o * 8, 8)], didx8)

        def g_start(q, j):
            pltpu.async_copy(u_hbm.at[sidx8.at[q]], rows.at[j], gsem[j])

        def g_wait(j):
            pltpu.make_async_copy(u_hbm.at[pl.ds(0, _K)], rows.at[j],
                                  gsem[j]).wait()

        def s_wait():
            pltpu.make_async_copy(u_hbm.at[pl.ds(0, _K)], rows.at[1], ss).wait()

        iload(0)
        g_start(0, 0)

        def body(o, t):
            for q in range(8):
                j = q % 2
                if q + 1 < 8:
                    if q == 0:
                        @pl.when(o > 0)
                        def _():
                            s_wait()
                    g_start(q + 1, 1 - j)
                g_wait(j)
                if q < 7:
                    pltpu.sync_copy(rows.at[j], acc.at[didx8.at[q]], add=True)
                else:
                    for v in range(_K // 16):
                        dtail[pl.ds(v * 16, 16)] = didx8[q, pl.ds(v * 16, 16)]
                    pltpu.async_copy(rows.at[j], acc.at[dtail], ss, add=True)
            o1 = jnp.minimum(o + 1, noct - 1)
            iload(o1)
            g_start(0, 0)
            return t

        lax.fori_loop(0, noct, body, 0)
        s_wait()
        g_wait(0)
        plsc.subcore_barrier()
        for t in range(SN // _K):
            pltpu.sync_copy(
                acc.at[pl.ds(s * SN + t * _K, _K)],
                out_hbm.at[c, pl.ds(s * SN + t * _K, _K)],
            )

    return k


def _dinv(degp_ref):
    deg = degp_ref[0, :] + degp_ref[1, :]
    return jnp.where(deg > 0, lax.rsqrt(deg), 0.0)


def _mm1_body(degp_ref, x_ref, w_ref, u_ref):
    dinv = _dinv(degp_ref)
    xw = jnp.dot(x_ref[...], w_ref[...], preferred_element_type=jnp.float32,
                 precision=lax.Precision.HIGHEST)
    u_ref[...] = xw * dinv[:, None]


def _mid_body(degp_ref, p_ref, b_ref, w_ref, u_ref):
    dinv = _dinv(degp_ref)
    ssum = p_ref[0] + p_ref[1]
    h = jnp.maximum(ssum * dinv[:, None] + b_ref[...], 0.0)
    hw = jnp.dot(h, w_ref[...], preferred_element_type=jnp.float32,
                 precision=lax.Precision.HIGHEST)
    u_ref[...] = hw * dinv[:, None]


def _fin_body(degp_ref, q_ref, b_ref, o_ref):
    dinv = _dinv(degp_ref)
    qsum = q_ref[0] + q_ref[1]
    o_ref[...] = qsum * dinv[:, None] + b_ref[...]


def _mm1(NP, DIN, DH):
    return pl.pallas_call(
        _mm1_body,
        grid=(NP // _MBLK,),
        in_specs=[
            pl.BlockSpec((_NC, _MBLK), lambda i: (0, i)),
            pl.BlockSpec((_MBLK, DIN), lambda i: (i, 0)),
            pl.BlockSpec((DIN, DH), lambda i: (0, 0)),
        ],
        out_specs=pl.BlockSpec((_MBLK, DH), lambda i: (i, 0)),
        out_shape=jax.ShapeDtypeStruct((NP, DH), jnp.float32),
    )


def _mid(NP, DH, D2):
    return pl.pallas_call(
        _mid_body,
        grid=(NP // _MBLK,),
        in_specs=[
            pl.BlockSpec((_NC, _MBLK), lambda i: (0, i)),
            pl.BlockSpec((_NC, _MBLK, DH), lambda i: (0, i, 0)),
            pl.BlockSpec((1, DH), lambda i: (0, 0)),
            pl.BlockSpec((DH, D2), lambda i: (0, 0)),
        ],
        out_specs=pl.BlockSpec((_MBLK, D2), lambda i: (i, 0)),
        out_shape=jax.ShapeDtypeStruct((NP, D2), jnp.float32),
    )


def _fin(NP, D2):
    return pl.pallas_call(
        _fin_body,
        grid=(NP // _MBLK,),
        in_specs=[
            pl.BlockSpec((_NC, _MBLK), lambda i: (0, i)),
            pl.BlockSpec((_NC, _MBLK, D2), lambda i: (0, i, 0)),
            pl.BlockSpec((1, D2), lambda i: (0, 0)),
        ],
        out_specs=pl.BlockSpec((_MBLK, D2), lambda i: (i, 0)),
        out_shape=jax.ShapeDtypeStruct((NP, D2), jnp.float32),
    )


def _impl(x, edge_index, W1, b1, Wmu, bmu, Wlv, blv):
    N, DIN = x.shape
    DH = W1.shape[1]
    DOUT = Wmu.shape[1]
    D2 = 2 * DOUT
    E = edge_index.shape[1]
    NW = _NC * _NS

    NP = math.ceil((N + 1) / (_NS * _K)) * (_NS * _K)
    NP = math.ceil(NP / _MBLK) * _MBLK
    Etot = E + N
    C = math.ceil(Etot / NW)
    nb8 = math.ceil(C / _K)
    nb8 = math.ceil(nb8 / 8) * 8
    SLAB = nb8 * _K
    noct = nb8 // 8

    def dump(n, off):
        return (N + (off + jnp.arange(n, dtype=jnp.int32)) % (NP - N)).astype(
            jnp.int32)

    def slabify(flat):
        a = jnp.concatenate([flat, dump(NW * C - Etot, 0)]).reshape(NW, C)
        b = dump(NW * (SLAB - C), 7).reshape(NW, SLAB - C)
        return jnp.concatenate([a, b], axis=1).reshape(NW, nb8, _K)

    sl = jnp.arange(N, dtype=jnp.int32)
    src2 = slabify(jnp.concatenate([edge_index[0], sl]))
    dst2 = slabify(jnp.concatenate([edge_index[1], sl]))
    xp = jnp.zeros((NP, DIN), jnp.float32).at[:N].set(x)

    degp = _deg_kernel(NP, nb8)(dst2)
    u1 = _mm1(NP, DIN, DH)(degp, xp, W1)
    P = _agg_kernel(NP, DH, noct)(u1, src2, dst2)
    Wcat = jnp.concatenate([Wmu, Wlv], axis=1)
    bcat = jnp.concatenate([bmu, blv]).reshape(1, D2)
    u2 = _mid(NP, DH, D2)(degp, P, b1.reshape(1, DH), Wcat)
    Q = _agg_kernel(NP, D2, noct)(u2, src2, dst2)
    out = _fin(NP, D2)(degp, Q, bcat)
    return out[:N, :DOUT], out[:N, DOUT:]


_jimpl = jax.jit(_impl)


def kernel(x, edge_index, W1, b1, Wmu, bmu, Wlv, blv):
    return _jimpl(x, edge_index, W1, b1, Wmu, bmu, Wlv, blv)


# speedup vs baseline: 34.0853x; 1.0981x over previous
import functools
import math

import jax
import jax.numpy as jnp
from jax import lax
from jax.experimental import pallas as pl
from jax.experimental.pallas import tpu as pltpu
from jax.experimental.pallas import tpu_sc as plsc

_NC = 2
_NS = 16
_K = 128
_MBLK = 1024


def _deg_kernel(NP, nb):
    NW = _NC * _NS
    SN = NP // _NS
    mesh = plsc.VectorSubcoreMesh(core_axis_name="c", subcore_axis_name="s")

    @functools.partial(
        pl.kernel,
        out_type=jax.ShapeDtypeStruct((_NC, NP), jnp.float32),
        mesh=mesh,
        scratch_types=[
            pltpu.VMEM((nb, _K), jnp.int32),
            pltpu.VMEM((_K,), jnp.float32),
            pltpu.VMEM((SN,), jnp.float32),
            pltpu.VMEM_SHARED((NP,), jnp.float32),
        ],
    )
    def k(dst_hbm, out_hbm, didx, ones, zbuf, acc):
        c = lax.axis_index("c")
        s = lax.axis_index("s")
        wid = c * _NS + s
        pltpu.sync_copy(dst_hbm.at[wid], didx)
        for j in range(_K // 16):
            ones[pl.ds(j * 16, 16)] = jnp.ones((16,), jnp.float32)

        def zb(i, t):
            zbuf[pl.ds(i * 16, 16)] = jnp.zeros((16,), jnp.float32)
            return t

        lax.fori_loop(0, SN // 16, zb, 0)
        pltpu.sync_copy(zbuf, acc.at[pl.ds(s * SN, SN)])
        plsc.subcore_barrier()

        def body(b, t):
            pltpu.sync_copy(ones, acc.at[didx.at[b]], add=True)
            return t

        lax.fori_loop(0, nb, body, 0)
        plsc.subcore_barrier()
        pltpu.sync_copy(acc.at[pl.ds(s * SN, SN)], out_hbm.at[c, pl.ds(s * SN, SN)])

    return k


def _agg_kernel(NP, D, noct, nbr):
    NW = _NC * _NS
    SN = NP // _NS
    mesh = plsc.VectorSubcoreMesh(core_axis_name="c", subcore_axis_name="s")

    @functools.partial(
        pl.kernel,
        out_type=jax.ShapeDtypeStruct((_NC, NP, D), jnp.float32),
        mesh=mesh,
        scratch_types=[
            pltpu.VMEM((8, _K), jnp.int32),
            pltpu.VMEM((8, _K), jnp.int32),
            pltpu.VMEM((_K,), jnp.int32),
            pltpu.VMEM((2, _K, D), jnp.float32),
            pltpu.VMEM_SHARED((NP, D), jnp.float32),
            pltpu.SemaphoreType.DMA,
            pltpu.SemaphoreType.DMA,
            pltpu.SemaphoreType.DMA,
        ],
    )
    def k(u_hbm, src_hbm, dst_hbm, out_hbm, sidx8, didx8, dtail, rows, acc,
          sg0, sg1, ss):
        c = lax.axis_index("c")
        s = lax.axis_index("s")
        wid = c * _NS + s

        def zr(i, t):
            for j in range(D // 16):
                rows[0, i, pl.ds(j * 16, 16)] = jnp.zeros((16,), jnp.float32)
            return t

        lax.fori_loop(0, _K, zr, 0)
        for t in range(SN // _K):
            pltpu.sync_copy(rows.at[0], acc.at[pl.ds(s * SN + t * _K, _K)])
        plsc.subcore_barrier()

        gsem = (sg0, sg1)

        def iload(o):
            pltpu.sync_copy(src_hbm.at[wid, pl.ds(o * 8, 8)], sidx8)
            pltpu.sync_copy(dst_hbm.at[wid, pl.ds(o * 8, 8)], didx8)

        def g_start(q, j):
            pltpu.async_copy(u_hbm.at[sidx8.at[q]], rows.at[j], gsem[j])

        def g_wait(j):
            pltpu.make_async_copy(u_hbm.at[pl.ds(0, _K)], rows.at[j],
                                  gsem[j]).wait()

        def s_wait():
            pltpu.make_async_copy(u_hbm.at[pl.ds(0, _K)], rows.at[1], ss).wait()

        iload(0)
        g_start(0, 0)

        def body(o, t):
            for q in range(8):
                j = q % 2
                live = o * 8 + q < nbr
                if q + 1 < 8:
                    if q == 0:
                        @pl.when(o > 0)
                        def _():
                            s_wait()
                    if 0 < q + 1 < 7:
                        @pl.when(o * 8 + q + 1 < nbr)
                        def _():
                            g_start(q + 1, 1 - j)
                    else:
                        g_start(q + 1, 1 - j)
                if q == 0 or q == 7:
                    g_wait(j)
                    if q == 0:
                        pltpu.sync_copy(rows.at[j], acc.at[didx8.at[q]],
                                        add=True)
                elif True:
                    @pl.when(live)
                    def _():
                        g_wait(j)
                        pltpu.sync_copy(rows.at[j], acc.at[didx8.at[q]],
                                        add=True)
                if q == 7:
                    for v in range(_K // 16):
                        dtail[pl.ds(v * 16, 16)] = didx8[q, pl.ds(v * 16, 16)]
                    pltpu.async_copy(rows.at[j], acc.at[dtail], ss, add=True)
            o1 = jnp.minimum(o + 1, noct - 1)
            iload(o1)
            g_start(0, 0)
            return t

        lax.fori_loop(0, noct, body, 0)
        s_wait()
        g_wait(0)
        plsc.subcore_barrier()
        for t in range(SN // _K):
            pltpu.sync_copy(
                acc.at[pl.ds(s * SN + t * _K, _K)],
                out_hbm.at[c, pl.ds(s * SN + t * _K, _K)],
            )

    return k


def _dinv(degp_ref):
    deg = degp_ref[0, :] + degp_ref[1, :]
    return jnp.where(deg > 0, lax.rsqrt(deg), 0.0)


def _mm1_body(degp_ref, x_ref, w_ref, u_ref):
    dinv = _dinv(degp_ref)
    xw = jnp.dot(x_ref[...], w_ref[...], preferred_element_type=jnp.float32)
    u_ref[...] = xw * dinv[:, None]


def _mid_body(degp_ref, p_ref, b_ref, w_ref, u_ref):
    dinv = _dinv(degp_ref)
    ssum = p_ref[0] + p_ref[1]
    h = jnp.maximum(ssum * dinv[:, None] + b_ref[...], 0.0)
    hw = jnp.dot(h, w_ref[...], preferred_element_type=jnp.float32)
    u_ref[...] = hw * dinv[:, None]


def _fin_body(degp_ref, q_ref, bmu_ref, blv_ref, mu_ref, lv_ref):
    dinv = _dinv(degp_ref)
    qsum = q_ref[0] + q_ref[1]
    half = qsum.shape[1] // 2
    mu_ref[...] = qsum[:, :half] * dinv[:, None] + bmu_ref[...]
    lv_ref[...] = qsum[:, half:] * dinv[:, None] + blv_ref[...]


def _mm1(NP, N, DIN, DH):
    return pl.pallas_call(
        _mm1_body,
        grid=(NP // _MBLK,),
        in_specs=[
            pl.BlockSpec((_NC, _MBLK), lambda i: (0, i)),
            pl.BlockSpec((_MBLK, DIN), lambda i: (i, 0)),
            pl.BlockSpec((DIN, DH), lambda i: (0, 0)),
        ],
        out_specs=pl.BlockSpec((_MBLK, DH), lambda i: (i, 0)),
        out_shape=jax.ShapeDtypeStruct((NP, DH), jnp.float32),
    )


def _mid(NP, DH, D2):
    return pl.pallas_call(
        _mid_body,
        grid=(NP // _MBLK,),
        in_specs=[
            pl.BlockSpec((_NC, _MBLK), lambda i: (0, i)),
            pl.BlockSpec((_NC, _MBLK, DH), lambda i: (0, i, 0)),
            pl.BlockSpec((1, DH), lambda i: (0, 0)),
            pl.BlockSpec((DH, D2), lambda i: (0, 0)),
        ],
        out_specs=pl.BlockSpec((_MBLK, D2), lambda i: (i, 0)),
        out_shape=jax.ShapeDtypeStruct((NP, D2), jnp.float32),
    )


def _fin(NP, N, D2):
    DOUT = D2 // 2
    return pl.pallas_call(
        _fin_body,
        grid=(NP // _MBLK,),
        in_specs=[
            pl.BlockSpec((_NC, _MBLK), lambda i: (0, i)),
            pl.BlockSpec((_NC, _MBLK, D2), lambda i: (0, i, 0)),
            pl.BlockSpec((1, DOUT), lambda i: (0, 0)),
            pl.BlockSpec((1, DOUT), lambda i: (0, 0)),
        ],
        out_specs=[
            pl.BlockSpec((_MBLK, DOUT), lambda i: (i, 0)),
            pl.BlockSpec((_MBLK, DOUT), lambda i: (i, 0)),
        ],
        out_shape=[
            jax.ShapeDtypeStruct((N, DOUT), jnp.float32),
            jax.ShapeDtypeStruct((N, DOUT), jnp.float32),
        ],
    )


def _impl(x, edge_index, W1, b1, Wmu, bmu, Wlv, blv):
    N, DIN = x.shape
    DH = W1.shape[1]
    DOUT = Wmu.shape[1]
    D2 = 2 * DOUT
    E = edge_index.shape[1]
    NW = _NC * _NS

    NP = math.ceil((N + 1) / (_NS * _K)) * (_NS * _K)
    NP = math.ceil(NP / _MBLK) * _MBLK
    Etot = E + N
    C = math.ceil(Etot / NW)
    nbr = math.ceil(C / _K)
    nb8 = math.ceil(nbr / 8) * 8
    SLAB = nb8 * _K
    noct = nb8 // 8

    def dump(n, off):
        return (N + (off + jnp.arange(n, dtype=jnp.int32)) % (NP - N)).astype(
            jnp.int32)

    def slabify(flat):
        a = jnp.concatenate([flat, dump(NW * C - Etot, 0)]).reshape(NW, C)
        b = dump(NW * (SLAB - C), 7).reshape(NW, SLAB - C)
        return jnp.concatenate([a, b], axis=1).reshape(NW, nb8, _K)

    sl = jnp.arange(N, dtype=jnp.int32)
    src2 = slabify(jnp.concatenate([edge_index[0], sl]))
    dst2 = slabify(jnp.concatenate([edge_index[1], sl]))

    degp = _deg_kernel(NP, nb8)(dst2)
    u1 = _mm1(NP, N, DIN, DH)(degp, x, W1)
    P = _agg_kernel(NP, DH, noct, nbr)(u1, src2, dst2)
    Wcat = jnp.concatenate([Wmu, Wlv], axis=1)
    u2 = _mid(NP, DH, D2)(degp, P, b1.reshape(1, DH), Wcat)
    Q = _agg_kernel(NP, D2, noct, nbr)(u2, src2, dst2)
    mu, lv = _fin(NP, N, D2)(degp, Q, bmu.reshape(1, DOUT), blv.reshape(1, DOUT))
    return mu, lv


_jimpl = jax.jit(_impl)


def kernel(x, edge_index, W1, b1, Wmu, bmu, Wlv, blv):
    return _jimpl(x, edge_index, W1, b1, Wmu, bmu, Wlv, blv)


# speedup vs baseline: 34.6924x; 1.0178x over previous
import functools
import math

import jax
import jax.numpy as jnp
from jax import lax
from jax.experimental import pallas as pl
from jax.experimental.pallas import tpu as pltpu
from jax.experimental.pallas import tpu_sc as plsc

_NC = 2
_NS = 16
_K = 128
_MBLK = 2048


def _deg_kernel(NP, nb):
    NW = _NC * _NS
    SN = NP // _NS
    mesh = plsc.VectorSubcoreMesh(core_axis_name="c", subcore_axis_name="s")

    @functools.partial(
        pl.kernel,
        out_type=jax.ShapeDtypeStruct((_NC, NP), jnp.float32),
        mesh=mesh,
        scratch_types=[
            pltpu.VMEM((nb, _K), jnp.int32),
            pltpu.VMEM((_K,), jnp.float32),
            pltpu.VMEM((SN,), jnp.float32),
            pltpu.VMEM_SHARED((NP,), jnp.float32),
        ],
    )
    def k(dst_hbm, out_hbm, didx, ones, zbuf, acc):
        c = lax.axis_index("c")
        s = lax.axis_index("s")
        wid = c * _NS + s
        pltpu.sync_copy(dst_hbm.at[wid], didx)
        for j in range(_K // 16):
            ones[pl.ds(j * 16, 16)] = jnp.ones((16,), jnp.float32)

        def zb(i, t):
            zbuf[pl.ds(i * 16, 16)] = jnp.zeros((16,), jnp.float32)
            return t

        lax.fori_loop(0, SN // 16, zb, 0)
        pltpu.sync_copy(zbuf, acc.at[pl.ds(s * SN, SN)])
        plsc.subcore_barrier()

        def body(b, t):
            pltpu.sync_copy(ones, acc.at[didx.at[b]], add=True)
            return t

        lax.fori_loop(0, nb, body, 0)
        plsc.subcore_barrier()
        pltpu.sync_copy(acc.at[pl.ds(s * SN, SN)], out_hbm.at[c, pl.ds(s * SN, SN)])

    return k


def _agg_kernel(NP, D, noct, nbr):
    NW = _NC * _NS
    SN = NP // _NS
    mesh = plsc.VectorSubcoreMesh(core_axis_name="c", subcore_axis_name="s")

    @functools.partial(
        pl.kernel,
        out_type=jax.ShapeDtypeStruct((_NC, NP, D), jnp.float32),
        mesh=mesh,
        scratch_types=[
            pltpu.VMEM((8, _K), jnp.int32),
            pltpu.VMEM((8, _K), jnp.int32),
            pltpu.VMEM((_K,), jnp.int32),
            pltpu.VMEM((2, _K, D), jnp.float32),
            pltpu.VMEM_SHARED((NP, D), jnp.float32),
            pltpu.SemaphoreType.DMA,
            pltpu.SemaphoreType.DMA,
            pltpu.SemaphoreType.DMA,
        ],
    )
    def k(u_hbm, src_hbm, dst_hbm, out_hbm, sidx8, didx8, dtail, rows, acc,
          sg0, sg1, ss):
        c = lax.axis_index("c")
        s = lax.axis_index("s")
        wid = c * _NS + s

        def zr(i, t):
            for j in range(D // 16):
                rows[0, i, pl.ds(j * 16, 16)] = jnp.zeros((16,), jnp.float32)
            return t

        lax.fori_loop(0, _K, zr, 0)
        for t in range(SN // _K):
            pltpu.sync_copy(rows.at[0], acc.at[pl.ds(s * SN + t * _K, _K)])
        plsc.subcore_barrier()

        gsem = (sg0, sg1)

        def iload(o):
            pltpu.sync_copy(src_hbm.at[wid, pl.ds(o * 8, 8)], sidx8)
            pltpu.sync_copy(dst_hbm.at[wid, pl.ds(o * 8, 8)], didx8)

        def g_start(q, j):
            pltpu.async_copy(u_hbm.at[sidx8.at[q]], rows.at[j], gsem[j])

        def g_wait(j):
            pltpu.make_async_copy(u_hbm.at[pl.ds(0, _K)], rows.at[j],
                                  gsem[j]).wait()

        def s_wait():
            pltpu.make_async_copy(u_hbm.at[pl.ds(0, _K)], rows.at[1], ss).wait()

        iload(0)
        g_start(0, 0)

        def body(o, t):
            for q in range(8):
                j = q % 2
                live = o * 8 + q < nbr
                if q + 1 < 8:
                    if q == 0:
                        @pl.when(o > 0)
                        def _():
                            s_wait()
                    if 0 < q + 1 < 7:
                        @pl.when(o * 8 + q + 1 < nbr)
                        def _():
                            g_start(q + 1, 1 - j)
                    else:
                        g_start(q + 1, 1 - j)
                if q == 0 or q == 7:
                    g_wait(j)
                    if q == 0:
                        pltpu.sync_copy(rows.at[j], acc.at[didx8.at[q]],
                                        add=True)
                elif True:
                    @pl.when(live)
                    def _():
                        g_wait(j)
                        pltpu.sync_copy(rows.at[j], acc.at[didx8.at[q]],
                                        add=True)
                if q == 7:
                    for v in range(_K // 16):
                        dtail[pl.ds(v * 16, 16)] = didx8[q, pl.ds(v * 16, 16)]
                    pltpu.async_copy(rows.at[j], acc.at[dtail], ss, add=True)
            o1 = jnp.minimum(o + 1, noct - 1)
            iload(o1)
            g_start(0, 0)
            return t

        lax.fori_loop(0, noct, body, 0)
        s_wait()
        g_wait(0)
        plsc.subcore_barrier()
        for t in range(SN // _K):
            pltpu.sync_copy(
                acc.at[pl.ds(s * SN + t * _K, _K)],
                out_hbm.at[c, pl.ds(s * SN + t * _K, _K)],
            )

    return k


def _dinv(degp_ref):
    deg = degp_ref[0, :] + degp_ref[1, :]
    return jnp.where(deg > 0, lax.rsqrt(deg), 0.0)


def _mm1_body(degp_ref, x_ref, w_ref, u_ref):
    dinv = _dinv(degp_ref)
    xw = jnp.dot(x_ref[...], w_ref[...], preferred_element_type=jnp.float32)
    u_ref[...] = xw * dinv[:, None]


def _mid_body(degp_ref, p_ref, b_ref, w_ref, u_ref):
    dinv = _dinv(degp_ref)
    ssum = p_ref[0] + p_ref[1]
    h = jnp.maximum(ssum * dinv[:, None] + b_ref[...], 0.0)
    hw = jnp.dot(h, w_ref[...], preferred_element_type=jnp.float32)
    u_ref[...] = hw * dinv[:, None]


def _fin_body(degp_ref, q_ref, bmu_ref, blv_ref, mu_ref, lv_ref):
    dinv = _dinv(degp_ref)
    qsum = q_ref[0] + q_ref[1]
    half = qsum.shape[1] // 2
    mu_ref[...] = qsum[:, :half] * dinv[:, None] + bmu_ref[...]
    lv_ref[...] = qsum[:, half:] * dinv[:, None] + blv_ref[...]


def _mm1(NP, N, DIN, DH):
    return pl.pallas_call(
        _mm1_body,
        grid=(NP // _MBLK,),
        in_specs=[
            pl.BlockSpec((_NC, _MBLK), lambda i: (0, i)),
            pl.BlockSpec((_MBLK, DIN), lambda i: (i, 0)),
            pl.BlockSpec((DIN, DH), lambda i: (0, 0)),
        ],
        out_specs=pl.BlockSpec((_MBLK, DH), lambda i: (i, 0)),
        out_shape=jax.ShapeDtypeStruct((NP, DH), jnp.float32),
    )


def _mid(NP, DH, D2):
    return pl.pallas_call(
        _mid_body,
        grid=(NP // _MBLK,),
        in_specs=[
            pl.BlockSpec((_NC, _MBLK), lambda i: (0, i)),
            pl.BlockSpec((_NC, _MBLK, DH), lambda i: (0, i, 0)),
            pl.BlockSpec((1, DH), lambda i: (0, 0)),
            pl.BlockSpec((DH, D2), lambda i: (0, 0)),
        ],
        out_specs=pl.BlockSpec((_MBLK, D2), lambda i: (i, 0)),
        out_shape=jax.ShapeDtypeStruct((NP, D2), jnp.float32),
    )


def _fin(NP, N, D2):
    DOUT = D2 // 2
    return pl.pallas_call(
        _fin_body,
        grid=(NP // _MBLK,),
        in_specs=[
            pl.BlockSpec((_NC, _MBLK), lambda i: (0, i)),
            pl.BlockSpec((_NC, _MBLK, D2), lambda i: (0, i, 0)),
            pl.BlockSpec((1, DOUT), lambda i: (0, 0)),
            pl.BlockSpec((1, DOUT), lambda i: (0, 0)),
        ],
        out_specs=[
            pl.BlockSpec((_MBLK, DOUT), lambda i: (i, 0)),
            pl.BlockSpec((_MBLK, DOUT), lambda i: (i, 0)),
        ],
        out_shape=[
            jax.ShapeDtypeStruct((N, DOUT), jnp.float32),
            jax.ShapeDtypeStruct((N, DOUT), jnp.float32),
        ],
    )


def _impl(x, edge_index, W1, b1, Wmu, bmu, Wlv, blv):
    N, DIN = x.shape
    DH = W1.shape[1]
    DOUT = Wmu.shape[1]
    D2 = 2 * DOUT
    E = edge_index.shape[1]
    NW = _NC * _NS

    NP = math.ceil((N + 1) / (_NS * _K)) * (_NS * _K)
    NP = math.ceil(NP / _MBLK) * _MBLK
    Etot = E + N
    C = math.ceil(Etot / NW)
    nbr = math.ceil(C / _K)
    nb8 = math.ceil(nbr / 8) * 8
    SLAB = nb8 * _K
    noct = nb8 // 8

    def dump(n, off):
        return (N + (off + jnp.arange(n, dtype=jnp.int32)) % (NP - N)).astype(
            jnp.int32)

    def slabify(flat):
        a = jnp.concatenate([flat, dump(NW * C - Etot, 0)]).reshape(NW, C)
        b = dump(NW * (SLAB - C), 7).reshape(NW, SLAB - C)
        return jnp.concatenate([a, b], axis=1).reshape(NW, nb8, _K)

    sl = jnp.arange(N, dtype=jnp.int32)
    src2 = slabify(jnp.concatenate([edge_index[0], sl]))
    dst2 = slabify(jnp.concatenate([edge_index[1], sl]))

    degp = _deg_kernel(NP, nb8)(dst2)
    u1 = _mm1(NP, N, DIN, DH)(degp, x, W1)
    P = _agg_kernel(NP, DH, noct, nbr)(u1, src2, dst2)
    Wcat = jnp.concatenate([Wmu, Wlv], axis=1)
    u2 = _mid(NP, DH, D2)(degp, P, b1.reshape(1, DH), Wcat)
    Q = _agg_kernel(NP, D2, noct, nbr)(u2, src2, dst2)
    mu, lv = _fin(NP, N, D2)(degp, Q, bmu.reshape(1, DOUT), blv.reshape(1, DOUT))
    return mu, lv


_jimpl = jax.jit(_impl)


def kernel(x, edge_index, W1, b1, Wmu, bmu, Wlv, blv):
    return _jimpl(x, edge_index, W1, b1, Wmu, bmu, Wlv, blv)


# speedup vs baseline: 38.6299x; 1.1135x over previous
import functools
import math

import jax
import jax.numpy as jnp
from jax import lax
from jax.experimental import pallas as pl
from jax.experimental.pallas import tpu as pltpu
from jax.experimental.pallas import tpu_sc as plsc

_NC = 2
_NS = 16
_K = 128
_MBLK = 2048


def _deg_kernel(NP, nb):
    NW = _NC * _NS
    SN = NP // _NS
    mesh = plsc.VectorSubcoreMesh(core_axis_name="c", subcore_axis_name="s")

    @functools.partial(
        pl.kernel,
        out_type=jax.ShapeDtypeStruct((_NC, NP), jnp.float32),
        mesh=mesh,
        scratch_types=[
            pltpu.VMEM((nb, _K), jnp.int32),
            pltpu.VMEM((_K,), jnp.float32),
            pltpu.VMEM((SN,), jnp.float32),
            pltpu.VMEM_SHARED((NP,), jnp.float32),
        ],
    )
    def k(dst_hbm, out_hbm, didx, ones, zbuf, acc):
        c = lax.axis_index("c")
        s = lax.axis_index("s")
        wid = c * _NS + s
        pltpu.sync_copy(dst_hbm.at[wid], didx)
        for j in range(_K // 16):
            ones[pl.ds(j * 16, 16)] = jnp.ones((16,), jnp.float32)

        def zb(i, t):
            zbuf[pl.ds(i * 16, 16)] = jnp.zeros((16,), jnp.float32)
            return t

        lax.fori_loop(0, SN // 16, zb, 0)
        pltpu.sync_copy(zbuf, acc.at[pl.ds(s * SN, SN)])
        plsc.subcore_barrier()

        def body(b, t):
            pltpu.sync_copy(ones, acc.at[didx.at[b]], add=True)
            return t

        lax.fori_loop(0, nb, body, 0)
        plsc.subcore_barrier()
        pltpu.sync_copy(acc.at[pl.ds(s * SN, SN)], out_hbm.at[c, pl.ds(s * SN, SN)])

    return k


def _agg_kernel(NP, D, noct, nbr):
    NW = _NC * _NS
    SN = NP // _NS
    mesh = plsc.VectorSubcoreMesh(core_axis_name="c", subcore_axis_name="s")

    @functools.partial(
        pl.kernel,
        out_type=jax.ShapeDtypeStruct((_NC, NP, D), jnp.float32),
        mesh=mesh,
        scratch_types=[
            pltpu.VMEM((2, 8, _K), jnp.int32),
            pltpu.VMEM((2, 8, _K), jnp.int32),
            pltpu.VMEM((_K,), jnp.int32),
            pltpu.VMEM((2, _K, D), jnp.float32),
            pltpu.VMEM_SHARED((NP, D), jnp.float32),
            pltpu.SemaphoreType.DMA,
            pltpu.SemaphoreType.DMA,
            pltpu.SemaphoreType.DMA,
            pltpu.SemaphoreType.DMA,
        ],
    )
    def k(u_hbm, src_hbm, dst_hbm, out_hbm, sidx8, didx8, dtail, rows, acc,
          sg0, sg1, ss, si):
        c = lax.axis_index("c")
        s = lax.axis_index("s")
        wid = c * _NS + s

        gsem = (sg0, sg1)

        def i_start(o, p):
            pltpu.async_copy(src_hbm.at[wid, pl.ds(o * 8, 8)], sidx8.at[p], si)
            pltpu.async_copy(dst_hbm.at[wid, pl.ds(o * 8, 8)], didx8.at[p], si)

        def i_wait():
            for _ in range(2):
                pltpu.make_async_copy(src_hbm.at[wid, pl.ds(0, 8)],
                                      sidx8.at[0], si).wait()

        def g_start(p, q, j):
            pltpu.async_copy(u_hbm.at[sidx8.at[p, q]], rows.at[j], gsem[j])

        def g_wait(j):
            pltpu.make_async_copy(u_hbm.at[pl.ds(0, _K)], rows.at[j],
                                  gsem[j]).wait()

        def s_wait():
            pltpu.make_async_copy(u_hbm.at[pl.ds(0, _K)], rows.at[1], ss).wait()

        i_start(0, 0)

        def zr(i, t):
            for j in range(D // 16):
                rows[0, i, pl.ds(j * 16, 16)] = jnp.zeros((16,), jnp.float32)
            return t

        lax.fori_loop(0, _K, zr, 0)
        for t in range(SN // _K):
            pltpu.sync_copy(rows.at[0], acc.at[pl.ds(s * SN + t * _K, _K)])
        plsc.subcore_barrier()

        i_wait()
        g_start(0, 0, 0)

        def body(o, t):
            po = o % 2
            o1 = jnp.minimum(o + 1, noct - 1)
            i_start(o1, 1 - po)
            for q in range(8):
                j = q % 2
                live = o * 8 + q < nbr
                if q + 1 < 8:
                    if q == 0:
                        @pl.when(o > 0)
                        def _():
                            s_wait()
                    if 0 < q + 1 < 7:
                        @pl.when(o * 8 + q + 1 < nbr)
                        def _():
                            g_start(po, q + 1, 1 - j)
                    else:
                        g_start(po, q + 1, 1 - j)
                if q == 0 or q == 7:
                    g_wait(j)
                    if q == 0:
                        pltpu.sync_copy(rows.at[j], acc.at[didx8.at[po, q]],
                                        add=True)
                elif True:
                    @pl.when(live)
                    def _():
                        g_wait(j)
                        pltpu.sync_copy(rows.at[j], acc.at[didx8.at[po, q]],
                                        add=True)
                if q == 7:
                    for v in range(_K // 16):
                        dtail[pl.ds(v * 16, 16)] = didx8[po, q,
                                                         pl.ds(v * 16, 16)]
                    pltpu.async_copy(rows.at[j], acc.at[dtail], ss, add=True)
            i_wait()
            g_start(1 - po, 0, 0)
            return t

        lax.fori_loop(0, noct, body, 0)
        s_wait()
        g_wait(0)
        plsc.subcore_barrier()
        for t in range(SN // _K):
            pltpu.sync_copy(
                acc.at[pl.ds(s * SN + t * _K, _K)],
                out_hbm.at[c, pl.ds(s * SN + t * _K, _K)],
            )

    return k


def _dinv(degp_ref):
    deg = degp_ref[0, :] + degp_ref[1, :]
    return jnp.where(deg > 0, lax.rsqrt(deg), 0.0)


def _mm1_body(degp_ref, x_ref, w_ref, u_ref):
    dinv = _dinv(degp_ref)
    xw = jnp.dot(x_ref[...], w_ref[...], preferred_element_type=jnp.float32)
    u_ref[...] = xw * dinv[:, None]


def _mid_body(degp_ref, p_ref, b_ref, w_ref, u_ref):
    dinv = _dinv(degp_ref)
    ssum = p_ref[0] + p_ref[1]
    h = jnp.maximum(ssum * dinv[:, None] + b_ref[...], 0.0)
    hw = jnp.dot(h, w_ref[...], preferred_element_type=jnp.float32)
    u_ref[...] = hw * dinv[:, None]


def _fin_body(degp_ref, q_ref, bmu_ref, blv_ref, mu_ref, lv_ref):
    dinv = _dinv(degp_ref)
    qsum = q_ref[0] + q_ref[1]
    half = qsum.shape[1] // 2
    mu_ref[...] = qsum[:, :half] * dinv[:, None] + bmu_ref[...]
    lv_ref[...] = qsum[:, half:] * dinv[:, None] + blv_ref[...]


def _mm1(NP, N, DIN, DH):
    return pl.pallas_call(
        _mm1_body,
        grid=(NP // _MBLK,),
        in_specs=[
            pl.BlockSpec((_NC, _MBLK), lambda i: (0, i)),
            pl.BlockSpec((_MBLK, DIN), lambda i: (i, 0)),
            pl.BlockSpec((DIN, DH), lambda i: (0, 0)),
        ],
        out_specs=pl.BlockSpec((_MBLK, DH), lambda i: (i, 0)),
        out_shape=jax.ShapeDtypeStruct((NP, DH), jnp.float32),
    )


def _mid(NP, DH, D2):
    return pl.pallas_call(
        _mid_body,
        grid=(NP // _MBLK,),
        in_specs=[
            pl.BlockSpec((_NC, _MBLK), lambda i: (0, i)),
            pl.BlockSpec((_NC, _MBLK, DH), lambda i: (0, i, 0)),
            pl.BlockSpec((1, DH), lambda i: (0, 0)),
            pl.BlockSpec((DH, D2), lambda i: (0, 0)),
        ],
        out_specs=pl.BlockSpec((_MBLK, D2), lambda i: (i, 0)),
        out_shape=jax.ShapeDtypeStruct((NP, D2), jnp.float32),
    )


def _fin(NP, N, D2):
    DOUT = D2 // 2
    return pl.pallas_call(
        _fin_body,
        grid=(NP // _MBLK,),
        in_specs=[
            pl.BlockSpec((_NC, _MBLK), lambda i: (0, i)),
            pl.BlockSpec((_NC, _MBLK, D2), lambda i: (0, i, 0)),
            pl.BlockSpec((1, DOUT), lambda i: (0, 0)),
            pl.BlockSpec((1, DOUT), lambda i: (0, 0)),
        ],
        out_specs=[
            pl.BlockSpec((_MBLK, DOUT), lambda i: (i, 0)),
            pl.BlockSpec((_MBLK, DOUT), lambda i: (i, 0)),
        ],
        out_shape=[
            jax.ShapeDtypeStruct((N, DOUT), jnp.float32),
            jax.ShapeDtypeStruct((N, DOUT), jnp.float32),
        ],
    )


def _impl(x, edge_index, W1, b1, Wmu, bmu, Wlv, blv):
    N, DIN = x.shape
    DH = W1.shape[1]
    DOUT = Wmu.shape[1]
    D2 = 2 * DOUT
    E = edge_index.shape[1]
    NW = _NC * _NS

    NP = math.ceil((N + 1) / (_NS * _K)) * (_NS * _K)
    NP = math.ceil(NP / _MBLK) * _MBLK
    Etot = E + N
    C = math.ceil(Etot / NW)
    nbr = math.ceil(C / _K)
    nb8 = math.ceil(nbr / 8) * 8
    SLAB = nb8 * _K
    noct = nb8 // 8

    def dump(n, off):
        return (N + (off + jnp.arange(n, dtype=jnp.int32)) % (NP - N)).astype(
            jnp.int32)

    def slabify(flat):
        a = jnp.concatenate([flat, dump(NW * C - Etot, 0)]).reshape(NW, C)
        b = dump(NW * (SLAB - C), 7).reshape(NW, SLAB - C)
        return jnp.concatenate([a, b], axis=1).reshape(NW, nb8, _K)

    sl = jnp.arange(N, dtype=jnp.int32)
    src2 = slabify(jnp.concatenate([edge_index[0], sl]))
    dst2 = slabify(jnp.concatenate([edge_index[1], sl]))

    degp = _deg_kernel(NP, nb8)(dst2)
    u1 = _mm1(NP, N, DIN, DH)(degp, x, W1)
    P = _agg_kernel(NP, DH, noct, nbr)(u1, src2, dst2)
    Wcat = jnp.concatenate([Wmu, Wlv], axis=1)
    u2 = _mid(NP, DH, D2)(degp, P, b1.reshape(1, DH), Wcat)
    Q = _agg_kernel(NP, D2, noct, nbr)(u2, src2, dst2)
    mu, lv = _fin(NP, N, D2)(degp, Q, bmu.reshape(1, DOUT), blv.reshape(1, DOUT))
    return mu, lv


_jimpl = jax.jit(_impl)


def kernel(x, edge_index, W1, b1, Wmu, bmu, Wlv, blv):
    return _jimpl(x, edge_index, W1, b1, Wmu, bmu, Wlv, blv)


# speedup vs baseline: 39.1268x; 1.0129x over previous
import functools
import math

import jax
import jax.numpy as jnp
from jax import lax
from jax.experimental import pallas as pl
from jax.experimental.pallas import tpu as pltpu
from jax.experimental.pallas import tpu_sc as plsc

_NC = 2
_NS = 16
_K = 128
_MBLK = 2048


def _deg_kernel(NP, nb):
    NW = _NC * _NS
    SN = NP // _NS
    mesh = plsc.VectorSubcoreMesh(core_axis_name="c", subcore_axis_name="s")

    @functools.partial(
        pl.kernel,
        out_type=jax.ShapeDtypeStruct((_NC, NP), jnp.float32),
        mesh=mesh,
        scratch_types=[
            pltpu.VMEM((nb, _K), jnp.int32),
            pltpu.VMEM((_K,), jnp.float32),
            pltpu.VMEM((SN,), jnp.float32),
            pltpu.VMEM_SHARED((NP,), jnp.float32),
            pltpu.SemaphoreType.DMA,
        ],
    )
    def k(dst_hbm, out_hbm, didx, ones, zbuf, acc, sd):
        c = lax.axis_index("c")
        s = lax.axis_index("s")
        wid = c * _NS + s
        pltpu.sync_copy(dst_hbm.at[wid], didx)
        for j in range(_K // 16):
            ones[pl.ds(j * 16, 16)] = jnp.ones((16,), jnp.float32)

        def zb(i, t):
            zbuf[pl.ds(i * 16, 16)] = jnp.zeros((16,), jnp.float32)
            return t

        lax.fori_loop(0, SN // 16, zb, 0)
        pltpu.sync_copy(zbuf, acc.at[pl.ds(s * SN, SN)])
        plsc.subcore_barrier()

        def body(o, t):
            for q in range(8):
                pltpu.async_copy(ones, acc.at[didx.at[o * 8 + q]], sd,
                                 add=True)
            for q in range(8):
                pltpu.make_async_copy(out_hbm.at[0, pl.ds(0, _K)], ones,
                                      sd).wait()
            return t

        lax.fori_loop(0, nb // 8, body, 0)
        plsc.subcore_barrier()
        pltpu.sync_copy(acc.at[pl.ds(s * SN, SN)], out_hbm.at[c, pl.ds(s * SN, SN)])

    return k


def _agg_kernel(NP, D, noct, nbr):
    NW = _NC * _NS
    SN = NP // _NS
    mesh = plsc.VectorSubcoreMesh(core_axis_name="c", subcore_axis_name="s")

    @functools.partial(
        pl.kernel,
        out_type=jax.ShapeDtypeStruct((_NC, NP, D), jnp.float32),
        mesh=mesh,
        scratch_types=[
            pltpu.VMEM((2, 8, _K), jnp.int32),
            pltpu.VMEM((2, 8, _K), jnp.int32),
            pltpu.VMEM((_K,), jnp.int32),
            pltpu.VMEM((2, _K, D), jnp.float32),
            pltpu.VMEM_SHARED((NP, D), jnp.float32),
            pltpu.SemaphoreType.DMA,
            pltpu.SemaphoreType.DMA,
            pltpu.SemaphoreType.DMA,
            pltpu.SemaphoreType.DMA,
        ],
    )
    def k(u_hbm, src_hbm, dst_hbm, out_hbm, sidx8, didx8, dtail, rows, acc,
          sg0, sg1, ss, si):
        c = lax.axis_index("c")
        s = lax.axis_index("s")
        wid = c * _NS + s

        gsem = (sg0, sg1)

        def i_start(o, p):
            pltpu.async_copy(src_hbm.at[wid, pl.ds(o * 8, 8)], sidx8.at[p], si)
            pltpu.async_copy(dst_hbm.at[wid, pl.ds(o * 8, 8)], didx8.at[p], si)

        def i_wait():
            for _ in range(2):
                pltpu.make_async_copy(src_hbm.at[wid, pl.ds(0, 8)],
                                      sidx8.at[0], si).wait()

        def g_start(p, q, j):
            pltpu.async_copy(u_hbm.at[sidx8.at[p, q]], rows.at[j], gsem[j])

        def g_wait(j):
            pltpu.make_async_copy(u_hbm.at[pl.ds(0, _K)], rows.at[j],
                                  gsem[j]).wait()

        def s_wait():
            pltpu.make_async_copy(u_hbm.at[pl.ds(0, _K)], rows.at[1], ss).wait()

        i_start(0, 0)

        def zr(i, t):
            for j in range(D // 16):
                rows[0, i, pl.ds(j * 16, 16)] = jnp.zeros((16,), jnp.float32)
            return t

        lax.fori_loop(0, _K, zr, 0)
        for t in range(SN // _K):
            pltpu.sync_copy(rows.at[0], acc.at[pl.ds(s * SN + t * _K, _K)])
        plsc.subcore_barrier()

        i_wait()
        g_start(0, 0, 0)

        def body(o, t):
            po = o % 2
            o1 = jnp.minimum(o + 1, noct - 1)
            i_start(o1, 1 - po)
            for q in range(8):
                j = q % 2
                live = o * 8 + q < nbr
                if q + 1 < 8:
                    if q == 0:
                        @pl.when(o > 0)
                        def _():
                            s_wait()
                    if 0 < q + 1 < 7:
                        @pl.when(o * 8 + q + 1 < nbr)
                        def _():
                            g_start(po, q + 1, 1 - j)
                    else:
                        g_start(po, q + 1, 1 - j)
                if q == 0 or q == 7:
                    g_wait(j)
                    if q == 0:
                        pltpu.sync_copy(rows.at[j], acc.at[didx8.at[po, q]],
                                        add=True)
                elif True:
                    @pl.when(live)
                    def _():
                        g_wait(j)
                        pltpu.sync_copy(rows.at[j], acc.at[didx8.at[po, q]],
                                        add=True)
                if q == 7:
                    for v in range(_K // 16):
                        dtail[pl.ds(v * 16, 16)] = didx8[po, q,
                                                         pl.ds(v * 16, 16)]
                    pltpu.async_copy(rows.at[j], acc.at[dtail], ss, add=True)
            i_wait()
            g_start(1 - po, 0, 0)
            return t

        lax.fori_loop(0, noct, body, 0)
        s_wait()
        g_wait(0)
        plsc.subcore_barrier()
        for t in range(SN // _K):
            pltpu.sync_copy(
                acc.at[pl.ds(s * SN + t * _K, _K)],
                out_hbm.at[c, pl.ds(s * SN + t * _K, _K)],
            )

    return k


def _dinv(degp_ref):
    deg = degp_ref[0, :] + degp_ref[1, :]
    return jnp.where(deg > 0, lax.rsqrt(deg), 0.0)


def _mm1_body(degp_ref, x_ref, w_ref, u_ref):
    dinv = _dinv(degp_ref)
    xw = jnp.dot(x_ref[...], w_ref[...], preferred_element_type=jnp.float32)
    u_ref[...] = xw * dinv[:, None]


def _mid_body(degp_ref, p_ref, b_ref, w_ref, u_ref):
    dinv = _dinv(degp_ref)
    ssum = p_ref[0] + p_ref[1]
    h = jnp.maximum(ssum * dinv[:, None] + b_ref[...], 0.0)
    hw = jnp.dot(h, w_ref[...], preferred_element_type=jnp.float32)
    u_ref[...] = hw * dinv[:, None]


def _fin_body(degp_ref, q_ref, bmu_ref, blv_ref, mu_ref, lv_ref):
    dinv = _dinv(degp_ref)
    qsum = q_ref[0] + q_ref[1]
    half = qsum.shape[1] // 2
    mu_ref[...] = qsum[:, :half] * dinv[:, None] + bmu_ref[...]
    lv_ref[...] = qsum[:, half:] * dinv[:, None] + blv_ref[...]


def _mm1(NP, N, DIN, DH):
    return pl.pallas_call(
        _mm1_body,
        grid=(NP // _MBLK,),
        in_specs=[
            pl.BlockSpec((_NC, _MBLK), lambda i: (0, i)),
            pl.BlockSpec((_MBLK, DIN), lambda i: (i, 0)),
            pl.BlockSpec((DIN, DH), lambda i: (0, 0)),
        ],
        out_specs=pl.BlockSpec((_MBLK, DH), lambda i: (i, 0)),
        out_shape=jax.ShapeDtypeStruct((NP, DH), jnp.float32),
    )


def _mid(NP, DH, D2):
    return pl.pallas_call(
        _mid_body,
        grid=(NP // _MBLK,),
        in_specs=[
            pl.BlockSpec((_NC, _MBLK), lambda i: (0, i)),
            pl.BlockSpec((_NC, _MBLK, DH), lambda i: (0, i, 0)),
            pl.BlockSpec((1, DH), lambda i: (0, 0)),
            pl.BlockSpec((DH, D2), lambda i: (0, 0)),
        ],
        out_specs=pl.BlockSpec((_MBLK, D2), lambda i: (i, 0)),
        out_shape=jax.ShapeDtypeStruct((NP, D2), jnp.float32),
    )


def _fin(NP, N, D2):
    DOUT = D2 // 2
    return pl.pallas_call(
        _fin_body,
        grid=(NP // _MBLK,),
        in_specs=[
            pl.BlockSpec((_NC, _MBLK), lambda i: (0, i)),
            pl.BlockSpec((_NC, _MBLK, D2), lambda i: (0, i, 0)),
            pl.BlockSpec((1, DOUT), lambda i: (0, 0)),
            pl.BlockSpec((1, DOUT), lambda i: (0, 0)),
        ],
        out_specs=[
            pl.BlockSpec((_MBLK, DOUT), lambda i: (i, 0)),
            pl.BlockSpec((_MBLK, DOUT), lambda i: (i, 0)),
        ],
        out_shape=[
            jax.ShapeDtypeStruct((N, DOUT), jnp.float32),
            jax.ShapeDtypeStruct((N, DOUT), jnp.float32),
        ],
    )


def _impl(x, edge_index, W1, b1, Wmu, bmu, Wlv, blv):
    N, DIN = x.shape
    DH = W1.shape[1]
    DOUT = Wmu.shape[1]
    D2 = 2 * DOUT
    E = edge_index.shape[1]
    NW = _NC * _NS

    NP = math.ceil((N + 1) / (_NS * _K)) * (_NS * _K)
    NP = math.ceil(NP / _MBLK) * _MBLK
    Etot = E + N
    C = math.ceil(Etot / NW)
    nbr = math.ceil(C / _K)
    nb8 = math.ceil(nbr / 8) * 8
    SLAB = nb8 * _K
    noct = nb8 // 8

    def dump(n, off):
        return (N + (off + jnp.arange(n, dtype=jnp.int32)) % (NP - N)).astype(
            jnp.int32)

    def slabify(flat):
        a = jnp.concatenate([flat, dump(NW * C - Etot, 0)]).reshape(NW, C)
        b = dump(NW * (SLAB - C), 7).reshape(NW, SLAB - C)
        return jnp.concatenate([a, b], axis=1).reshape(NW, nb8, _K)

    sl = jnp.arange(N, dtype=jnp.int32)
    src2 = slabify(jnp.concatenate([edge_index[0], sl]))
    dst2 = slabify(jnp.concatenate([edge_index[1], sl]))

    degp = _deg_kernel(NP, nb8)(dst2)
    u1 = _mm1(NP, N, DIN, DH)(degp, x, W1)
    P = _agg_kernel(NP, DH, noct, nbr)(u1, src2, dst2)
    Wcat = jnp.concatenate([Wmu, Wlv], axis=1)
    u2 = _mid(NP, DH, D2)(degp, P, b1.reshape(1, DH), Wcat)
    Q = _agg_kernel(NP, D2, noct, nbr)(u2, src2, dst2)
    mu, lv = _fin(NP, N, D2)(degp, Q, bmu.reshape(1, DOUT), blv.reshape(1, DOUT))
    return mu, lv


_jimpl = jax.jit(_impl)


def kernel(x, edge_index, W1, b1, Wmu, bmu, Wlv, blv):
    return _jimpl(x, edge_index, W1, b1, Wmu, bmu, Wlv, blv)
